# Initial kernel scaffold; baseline (speedup 1.0000x reference)
import jax
import jax.numpy as jnp
from jax.experimental import pallas as pl


def kernel(x):
    raise NotImplementedError("write your pallas kernel here")


# speedup vs baseline: 2.5448x; 2.5448x over previous
import jax
import jax.numpy as jnp
from jax.experimental import pallas as pl
from jax.experimental.pallas import tpu as pltpu

S_BLK = 512
F_BLK = 1024


def _body(x_ref, o_ref, carry):
    s = pl.program_id(2)

    @pl.when(s == 0)
    def _():
        carry[...] = jnp.zeros_like(carry)

    blk = x_ref[0]
    acc = blk
    d = 1
    while d < S_BLK:
        zeros = jnp.zeros((d, acc.shape[1]), acc.dtype)
        acc = acc + jnp.concatenate([zeros, acc[:-d]], axis=0)
        d *= 2
    out = carry[...] + acc
    o_ref[0] = out
    carry[...] = out[-1:, :]


def kernel(x):
    B, S, F = x.shape
    grid = (B, F // F_BLK, S // S_BLK)
    return pl.pallas_call(
        _body,
        grid=grid,
        in_specs=[pl.BlockSpec((1, S_BLK, F_BLK), lambda b, f, s: (b, s, f))],
        out_specs=pl.BlockSpec((1, S_BLK, F_BLK), lambda b, f, s: (b, s, f)),
        out_shape=jax.ShapeDtypeStruct(x.shape, x.dtype),
        scratch_shapes=[pltpu.VMEM((1, F_BLK), jnp.float32)],
    )(x)


# speedup vs baseline: 2.6108x; 1.0259x over previous
import functools

import jax
import jax.numpy as jnp
from jax import lax
from jax.experimental import pallas as pl
from jax.experimental.pallas import tpu as pltpu
from jax.experimental.pallas import tpu_sc as plsc

B, S, F = 2, 8192, 2048
T = 128
FB = 128
NV = FB // 16
G = S // T
NC, NS = 2, 16
NFBLK = F // FB


def _compute_block(buf, cs):
    def sbody(s, cs):
        out = []
        for j in range(NV):
            c = cs[j] + buf[s, j * 16:(j + 1) * 16]
            buf[s, j * 16:(j + 1) * 16] = c
            out.append(c)
        return tuple(out)

    return lax.fori_loop(0, T, sbody, cs)


def _scan_body(x_hbm, o_hbm, buf0, buf1, ld0, ld1, st0, st1):
    wid = lax.axis_index("s") * NC + lax.axis_index("c")
    b = wid // NFBLK
    f0 = (wid % NFBLK) * FB
    bufs = (buf0, buf1)
    lds = (ld0, ld1)
    sts = (st0, st1)

    def load(g, k):
        pltpu.make_async_copy(
            x_hbm.at[b, pl.ds(g * T, T), pl.ds(f0, FB)], bufs[k], lds[k]
        ).start()

    def store_start(g, k):
        pltpu.make_async_copy(
            bufs[k], o_hbm.at[b, pl.ds(g * T, T), pl.ds(f0, FB)], sts[k]
        ).start()

    def store_wait(g, k):
        pltpu.make_async_copy(
            bufs[k], o_hbm.at[b, pl.ds(g * T, T), pl.ds(f0, FB)], sts[k]
        ).wait()

    load(0, 0)
    czero = jnp.zeros((16,), jnp.float32)

    def outer(i, cs):
        for k in (0, 1):
            g = 2 * i + k
            o = 1 - k

            @pl.when((g >= 1) & (g + 1 < G))
            def _():
                store_wait(g - 1, o)

            @pl.when(g + 1 < G)
            def _():
                load(g + 1, o)

            pltpu.make_async_copy(
                x_hbm.at[b, pl.ds(g * T, T), pl.ds(f0, FB)], bufs[k], lds[k]
            ).wait()
            cs = _compute_block(bufs[k], cs)
            store_start(g, k)
        return cs

    lax.fori_loop(0, G // 2, outer, (czero,) * NV)
    store_wait(G - 2, 0)
    store_wait(G - 1, 1)


def kernel(x):

    mesh = plsc.VectorSubcoreMesh(core_axis_name="c", subcore_axis_name="s")

    scan = functools.partial(
        pl.kernel,
        mesh=mesh,
        out_type=jax.ShapeDtypeStruct((B, S, F), jnp.float32),
        scratch_types=[
            pltpu.VMEM((T, FB), jnp.float32),
            pltpu.VMEM((T, FB), jnp.float32),
            pltpu.SemaphoreType.DMA,
            pltpu.SemaphoreType.DMA,
            pltpu.SemaphoreType.DMA,
            pltpu.SemaphoreType.DMA,
        ],
    )(_scan_body)

    return scan(x)


# speedup vs baseline: 2.7935x; 1.0700x over previous
import functools

import jax
import jax.numpy as jnp
from jax import lax
from jax.experimental import pallas as pl
from jax.experimental.pallas import tpu as pltpu
from jax.experimental.pallas import tpu_sc as plsc

B, S, F = 2, 8192, 2048
T = 256
FB = 128
NV = FB // 16
G = S // T
NC, NS = 2, 16
NFBLK = F // FB


def _compute_block(buf, cs):
    def sbody(s, cs):
        out = []
        for j in range(NV):
            c = cs[j] + buf[s, j * 16:(j + 1) * 16]
            buf[s, j * 16:(j + 1) * 16] = c
            out.append(c)
        return tuple(out)

    return lax.fori_loop(0, T, sbody, cs)


def _scan_body(x_hbm, o_hbm, buf0, buf1, ld0, ld1, st0, st1):
    wid = lax.axis_index("s") * NC + lax.axis_index("c")
    b = wid // NFBLK
    f0 = (wid % NFBLK) * FB
    bufs = (buf0, buf1)
    lds = (ld0, ld1)
    sts = (st0, st1)

    def load(g, k):
        pltpu.make_async_copy(
            x_hbm.at[b, pl.ds(g * T, T), pl.ds(f0, FB)], bufs[k], lds[k]
        ).start()

    def store_start(g, k):
        pltpu.make_async_copy(
            bufs[k], o_hbm.at[b, pl.ds(g * T, T), pl.ds(f0, FB)], sts[k]
        ).start()

    def store_wait(g, k):
        pltpu.make_async_copy(
            bufs[k], o_hbm.at[b, pl.ds(g * T, T), pl.ds(f0, FB)], sts[k]
        ).wait()

    load(0, 0)
    czero = jnp.zeros((16,), jnp.float32)

    def outer(i, cs):
        for k in (0, 1):
            g = 2 * i + k
            o = 1 - k

            @pl.when((g >= 1) & (g + 1 < G))
            def _():
                store_wait(g - 1, o)

            @pl.when(g + 1 < G)
            def _():
                load(g + 1, o)

            pltpu.make_async_copy(
                x_hbm.at[b, pl.ds(g * T, T), pl.ds(f0, FB)], bufs[k], lds[k]
            ).wait()
            cs = _compute_block(bufs[k], cs)
            store_start(g, k)
        return cs

    lax.fori_loop(0, G // 2, outer, (czero,) * NV)
    store_wait(G - 2, 0)
    store_wait(G - 1, 1)


def kernel(x):

    mesh = plsc.VectorSubcoreMesh(core_axis_name="c", subcore_axis_name="s")

    scan = functools.partial(
        pl.kernel,
        mesh=mesh,
        out_type=jax.ShapeDtypeStruct((B, S, F), jnp.float32),
        scratch_types=[
            pltpu.VMEM((T, FB), jnp.float32),
            pltpu.VMEM((T, FB), jnp.float32),
            pltpu.SemaphoreType.DMA,
            pltpu.SemaphoreType.DMA,
            pltpu.SemaphoreType.DMA,
            pltpu.SemaphoreType.DMA,
        ],
    )(_scan_body)

    return scan(x)
